# baseline (speedup 1.0000x reference)
import functools

import jax
import jax.numpy as jnp
from jax import lax
from jax.experimental import pallas as pl
from jax.experimental.pallas import tpu as pltpu
from jax.experimental.pallas import tpu_sc as plsc

B = 2
HF = 64
WF = 64
C = 256
N_ROIS = 512
POOLED = 7
SPP = 4
SCALE = 0.0625
NBINS = POOLED * POOLED
ROI_STRIDE = 56

NUM_TILES = 32
ROIS_PER_TILE = N_ROIS // NUM_TILES

f32 = jnp.float32
i32 = jnp.int32


def _recip04(c):
    return jnp.where(c == 1.0, f32(1.0),
           jnp.where(c == 2.0, f32(0.5),
           jnp.where(c == 3.0, f32(1.0 / 3.0),
           jnp.where(c == 4.0, f32(0.25), f32(0.0)))))


def _floor_i32(x):
    ti = x.astype(i32)
    return ti - (ti.astype(f32) > x).astype(i32)


def _round_half_even(x):
    ti = _floor_i32(x)
    t = ti.astype(f32)
    f = x - t
    odd = (ti % 2) == 1
    return jnp.where(f > 0.5, t + 1.0, jnp.where(f < 0.5, t, jnp.where(odd, t + 1.0, t)))


NSLOT = 8


def _sc_pool_body(fm_hbm, rois_hbm, off_hbm, out_hbm,
                  rois_v, off_v, patch_v, idx0_v, idx1_v, idx2_v, idx3_v,
                  idx4_v, idx5_v, idx6_v, idx7_v,
                  a_v, out_roi_v, sem0, sem1, sem2, sem3, sem4, sem5, sem6, sem7):
    sem_list = (sem0, sem1, sem2, sem3, sem4, sem5, sem6, sem7)
    idx_refs = (idx0_v, idx1_v, idx2_v, idx3_v, idx4_v, idx5_v, idx6_v, idx7_v)
    wid = lax.axis_index("s") * 2 + lax.axis_index("c")
    roi0 = wid * ROIS_PER_TILE

    pltpu.sync_copy(rois_hbm.at[pl.ds(roi0, ROIS_PER_TILE)], rois_v)
    pltpu.sync_copy(off_hbm.at[pl.ds(roi0, ROIS_PER_TILE)], off_v)

    io = lax.iota(i32, 16)
    khi = lax.shift_right_logical(io, 2)
    kwi = lax.bitwise_and(io, 3)
    khf = khi.astype(f32)
    kwf = kwi.astype(f32)
    lane_off = khi * WF + kwi

    def slot_refs(si):
        return (fm_hbm.at[idx_refs[si]],
                patch_v.at[pl.ds(si * 16, 16)])

    def reduce_store(s, binidx):
        av = a_v[s, pl.ds(0, 16)]
        a_s = [av[p] for p in range(16)]
        base = s * 16
        for cvb in range(16):
            col = pl.ds(cvb * 16, 16)
            s0 = a_s[0] * patch_v[base, col]
            s1 = a_s[1] * patch_v[base + 1, col]
            s2 = a_s[2] * patch_v[base + 2, col]
            s3 = a_s[3] * patch_v[base + 3, col]
            for p in range(4, 16, 4):
                s0 = s0 + a_s[p] * patch_v[base + p, col]
                s1 = s1 + a_s[p + 1] * patch_v[base + p + 1, col]
                s2 = s2 + a_s[p + 2] * patch_v[base + p + 2, col]
                s3 = s3 + a_s[p + 3] * patch_v[base + p + 3, col]
            out_roi_v[binidx, col] = (s0 + s1) + (s2 + s3)

    z16v = jnp.zeros((16,), f32)
    for pr in range(NBINS, ROI_STRIDE):
        for cvb in range(16):
            out_roi_v[pr, pl.ds(cvb * 16, 16)] = z16v

    def roi_body(r, carry0):
        rv = rois_v[r, pl.ds(0, 16)]
        b_i = rv[0].astype(i32)
        x1 = _round_half_even(rv[1])
        y1 = _round_half_even(rv[2])
        x2 = _round_half_even(rv[3])
        y2 = _round_half_even(rv[4])
        rsw = x1 * SCALE - 0.5
        rsh = y1 * SCALE - 0.5
        rew = (x2 + 1.0) * SCALE - 0.5
        reh = (y2 + 1.0) * SCALE - 0.5
        roi_w = jnp.maximum(rew - rsw, 0.1)
        roi_h = jnp.maximum(reh - rsh, 0.1)
        bin_w = roi_w * f32(1.0 / 7.0)
        bin_h = roi_h * f32(1.0 / 7.0)
        sub_w = bin_w * 0.25
        sub_h = bin_h * 0.25
        base_b = b_i * (HF * WF)

        def bin_step(t, carry):
            ph, pw = carry
            binidx = ph * 7 + pw
            tx = off_v[r, pl.ds(binidx, 16)][0]
            ty = off_v[r, pl.ds(49 + binidx, 16)][0]
            pwf = pw.astype(f32)
            phf = ph.astype(f32)
            wstart = pwf * bin_w + rsw + tx * roi_w
            hstart = phf * bin_h + rsh + ty * roi_h

            w0i = jnp.minimum(_floor_i32(jnp.minimum(jnp.maximum(wstart, 0.0), 63.0)), 60)
            h0i = jnp.minimum(_floor_i32(jnp.minimum(jnp.maximum(hstart, 0.0), 63.0)), 60)
            w0f = w0i.astype(f32)
            h0f = h0i.astype(f32)

            idxvec = (base_b + h0i * WF + w0i) + lane_off
            slot = lax.bitwise_and(t, NSLOT - 1)
            for si in range(NSLOT):
                @pl.when(slot == si)
                def _(si=si):
                    idx_refs[si][...] = idxvec
                    src_r, dst_r = slot_refs(si)
                    pltpu.async_copy(src_r, dst_r, sem_list[si])

            Ww = jnp.zeros((16,), f32)
            Wh = jnp.zeros((16,), f32)
            cw = f32(0.0)
            ch = f32(0.0)
            for i in range(SPP):
                w = wstart + i * sub_w
                mw = jnp.where((w >= -0.5) & (w <= 63.5), f32(1.0), f32(0.0))
                wc = jnp.minimum(jnp.maximum(w, 0.0), 63.0)
                Ww = Ww + mw * jnp.maximum(1.0 - jnp.abs(wc - (w0f + kwf)), 0.0)
                cw = cw + mw
                h = hstart + i * sub_h
                mh = jnp.where((h >= -0.5) & (h <= 63.5), f32(1.0), f32(0.0))
                hc = jnp.minimum(jnp.maximum(h, 0.0), 63.0)
                Wh = Wh + mh * jnp.maximum(1.0 - jnp.abs(hc - (h0f + khf)), 0.0)
                ch = ch + mh

            scale = _recip04(cw) * _recip04(ch)
            a_v[slot, pl.ds(0, 16)] = Wh * Ww * scale

            s3 = lax.bitwise_and(t + 1, NSLOT - 1)
            live = t >= NSLOT - 1
            for si in range(NSLOT):
                @pl.when(live & (s3 == si))
                def _(si=si):
                    src_r, dst_r = slot_refs(si)
                    pltpu.make_async_copy(src_r, dst_r, sem_list[si]).wait()

            @pl.when(live)
            def _():
                reduce_store(s3, t - (NSLOT - 1))

            pw1 = pw + 1
            wrap = pw1 == 7
            return (jnp.where(wrap, ph + 1, ph), jnp.where(wrap, 0, pw1))

        lax.fori_loop(0, NBINS, bin_step, (i32(0), i32(0)))
        for j in range(NBINS - (NSLOT - 1), NBINS):
            si = j & (NSLOT - 1)
            src_r, dst_r = slot_refs(si)
            pltpu.make_async_copy(src_r, dst_r, sem_list[si]).wait()
            reduce_store(si, j)

        pltpu.sync_copy(out_roi_v, out_hbm.at[pl.ds((roi0 + r) * ROI_STRIDE, ROI_STRIDE)])
        return carry0

    lax.fori_loop(0, ROIS_PER_TILE, roi_body, 0)


def _make_sc_pool():
    mesh = plsc.VectorSubcoreMesh(core_axis_name="c", subcore_axis_name="s")
    return pl.kernel(
        _sc_pool_body,
        mesh=mesh,
        out_type=jax.ShapeDtypeStruct((N_ROIS * ROI_STRIDE, C), f32),
        scratch_types=[
            pltpu.VMEM((ROIS_PER_TILE, 16), f32),
            pltpu.VMEM((ROIS_PER_TILE, 128), f32),
            pltpu.VMEM((NSLOT * 16, C), f32),
            pltpu.VMEM((16,), i32),
            pltpu.VMEM((16,), i32),
            pltpu.VMEM((16,), i32),
            pltpu.VMEM((16,), i32),
            pltpu.VMEM((16,), i32),
            pltpu.VMEM((16,), i32),
            pltpu.VMEM((16,), i32),
            pltpu.VMEM((16,), i32),
            pltpu.VMEM((NSLOT, 16), f32),
            pltpu.VMEM((ROI_STRIDE, C), f32),
            pltpu.SemaphoreType.DMA,
            pltpu.SemaphoreType.DMA,
            pltpu.SemaphoreType.DMA,
            pltpu.SemaphoreType.DMA,
            pltpu.SemaphoreType.DMA,
            pltpu.SemaphoreType.DMA,
            pltpu.SemaphoreType.DMA,
            pltpu.SemaphoreType.DMA,
        ],
    )


MM_K = ROI_STRIDE * C


def _mm_body(a_ref, w_ref, b_ref, o_ref):
    o_ref[...] = jnp.dot(a_ref[...], w_ref[...],
                         preferred_element_type=f32) + b_ref[...]


def _offset_matmul(a, w_pad, b_pad):
    return pl.pallas_call(
        _mm_body,
        out_shape=jax.ShapeDtypeStruct((N_ROIS, 128), f32),
    )(a, w_pad, b_pad)


def kernel(featuremap, rois, W_fc, b_fc):
    fm_rows = featuremap.reshape(B * HF * WF, C)
    rois_pad = jnp.pad(rois, ((0, 0), (0, 11)))
    w_big = jnp.zeros((ROI_STRIDE, C, 128), f32).at[:NBINS, :, :98].set(
        W_fc.reshape(NBINS, C, 98)).reshape(MM_K, 128)
    b_pad = jnp.pad(b_fc, ((0, 30),)).reshape(1, 128)

    sc_pool = _make_sc_pool()

    off0 = jnp.zeros((N_ROIS, 128), f32)
    out1 = sc_pool(fm_rows, rois_pad, off0)
    off = _offset_matmul(out1.reshape(N_ROIS, MM_K), w_big, b_pad)
    out2 = sc_pool(fm_rows, rois_pad, off)
    return out2.reshape(N_ROIS, ROI_STRIDE, C)[:, :NBINS].reshape(
        N_ROIS, POOLED, POOLED, C)


# speedup vs baseline: 1.0488x; 1.0488x over previous
import functools

import jax
import jax.numpy as jnp
from jax import lax
from jax.experimental import pallas as pl
from jax.experimental.pallas import tpu as pltpu
from jax.experimental.pallas import tpu_sc as plsc

B = 2
HF = 64
WF = 64
C = 256
N_ROIS = 512
POOLED = 7
SPP = 4
SCALE = 0.0625
NBINS = POOLED * POOLED
ROI_STRIDE = 56

NUM_TILES = 32
ROIS_PER_TILE = N_ROIS // NUM_TILES

f32 = jnp.float32
i32 = jnp.int32


def _recip04(c):
    return jnp.where(c == 1.0, f32(1.0),
           jnp.where(c == 2.0, f32(0.5),
           jnp.where(c == 3.0, f32(1.0 / 3.0),
           jnp.where(c == 4.0, f32(0.25), f32(0.0)))))


def _floor_i32(x):
    ti = x.astype(i32)
    return ti - (ti.astype(f32) > x).astype(i32)


def _round_half_even(x):
    ti = _floor_i32(x)
    t = ti.astype(f32)
    f = x - t
    odd = (ti % 2) == 1
    return jnp.where(f > 0.5, t + 1.0, jnp.where(f < 0.5, t, jnp.where(odd, t + 1.0, t)))


NSLOT = 4


def _sc_pool_body(fm_hbm, rois_hbm, off_hbm, out_hbm,
                  rois_v, off_v, patch_v, idx0_v, idx1_v, idx2_v, idx3_v,
                  a_v, out_roi_v, sem0, sem1, sem2, sem3):
    sem_list = (sem0, sem1, sem2, sem3)
    idx_refs = (idx0_v, idx1_v, idx2_v, idx3_v)
    wid = lax.axis_index("s") * 2 + lax.axis_index("c")
    roi0 = wid * ROIS_PER_TILE

    pltpu.sync_copy(rois_hbm.at[pl.ds(roi0, ROIS_PER_TILE)], rois_v)
    pltpu.sync_copy(off_hbm.at[pl.ds(roi0, ROIS_PER_TILE)], off_v)

    io = lax.iota(i32, 16)
    khi = lax.shift_right_logical(io, 2)
    kwi = lax.bitwise_and(io, 3)
    khf = khi.astype(f32)
    kwf = kwi.astype(f32)
    lane_off = khi * WF + kwi

    def slot_refs(si):
        return (fm_hbm.at[idx_refs[si]],
                patch_v.at[pl.ds(si * 16, 16)])

    def reduce_store(s, binidx):
        av = a_v[s, pl.ds(0, 16)]
        a_s = [av.at[jnp.full((16,), p, i32)].get(mode="promise_in_bounds")
               for p in range(16)]
        base = s * 16
        for cvb in range(16):
            col = pl.ds(cvb * 16, 16)
            s0 = a_s[0] * patch_v[base, col]
            s1 = a_s[1] * patch_v[base + 1, col]
            s2 = a_s[2] * patch_v[base + 2, col]
            s3 = a_s[3] * patch_v[base + 3, col]
            for p in range(4, 16, 4):
                s0 = s0 + a_s[p] * patch_v[base + p, col]
                s1 = s1 + a_s[p + 1] * patch_v[base + p + 1, col]
                s2 = s2 + a_s[p + 2] * patch_v[base + p + 2, col]
                s3 = s3 + a_s[p + 3] * patch_v[base + p + 3, col]
            out_roi_v[binidx, col] = (s0 + s1) + (s2 + s3)

    z16v = jnp.zeros((16,), f32)
    for pr in range(NBINS, ROI_STRIDE):
        for cvb in range(16):
            out_roi_v[pr, pl.ds(cvb * 16, 16)] = z16v

    def roi_body(r, carry0):
        rv = rois_v[r, pl.ds(0, 16)]
        b_i = rv[0].astype(i32)
        x1 = _round_half_even(rv[1])
        y1 = _round_half_even(rv[2])
        x2 = _round_half_even(rv[3])
        y2 = _round_half_even(rv[4])
        rsw = x1 * SCALE - 0.5
        rsh = y1 * SCALE - 0.5
        rew = (x2 + 1.0) * SCALE - 0.5
        reh = (y2 + 1.0) * SCALE - 0.5
        roi_w = jnp.maximum(rew - rsw, 0.1)
        roi_h = jnp.maximum(reh - rsh, 0.1)
        bin_w = roi_w * f32(1.0 / 7.0)
        bin_h = roi_h * f32(1.0 / 7.0)
        sub_w = bin_w * 0.25
        sub_h = bin_h * 0.25
        base_b = b_i * (HF * WF)

        def bin_step(t, carry):
            ph, pw = carry
            binidx = ph * 7 + pw
            tx = off_v[r, pl.ds(binidx, 16)][0]
            ty = off_v[r, pl.ds(49 + binidx, 16)][0]
            pwf = pw.astype(f32)
            phf = ph.astype(f32)
            wstart = pwf * bin_w + rsw + tx * roi_w
            hstart = phf * bin_h + rsh + ty * roi_h

            w0i = jnp.minimum(_floor_i32(jnp.minimum(jnp.maximum(wstart, 0.0), 63.0)), 60)
            h0i = jnp.minimum(_floor_i32(jnp.minimum(jnp.maximum(hstart, 0.0), 63.0)), 60)
            w0f = w0i.astype(f32)
            h0f = h0i.astype(f32)

            idxvec = (base_b + h0i * WF + w0i) + lane_off
            slot = lax.bitwise_and(t, NSLOT - 1)
            for si in range(NSLOT):
                @pl.when(slot == si)
                def _(si=si):
                    idx_refs[si][...] = idxvec
                    src_r, dst_r = slot_refs(si)
                    pltpu.async_copy(src_r, dst_r, sem_list[si])

            Ww = jnp.zeros((16,), f32)
            Wh = jnp.zeros((16,), f32)
            cw = f32(0.0)
            ch = f32(0.0)
            for i in range(SPP):
                w = wstart + i * sub_w
                mw = jnp.where((w >= -0.5) & (w <= 63.5), f32(1.0), f32(0.0))
                wc = jnp.minimum(jnp.maximum(w, 0.0), 63.0)
                Ww = Ww + mw * jnp.maximum(1.0 - jnp.abs(wc - (w0f + kwf)), 0.0)
                cw = cw + mw
                h = hstart + i * sub_h
                mh = jnp.where((h >= -0.5) & (h <= 63.5), f32(1.0), f32(0.0))
                hc = jnp.minimum(jnp.maximum(h, 0.0), 63.0)
                Wh = Wh + mh * jnp.maximum(1.0 - jnp.abs(hc - (h0f + khf)), 0.0)
                ch = ch + mh

            scale = _recip04(cw) * _recip04(ch)
            a_v[slot, pl.ds(0, 16)] = Wh * Ww * scale

            s3 = lax.bitwise_and(t + 1, NSLOT - 1)
            live = t >= NSLOT - 1
            for si in range(NSLOT):
                @pl.when(live & (s3 == si))
                def _(si=si):
                    src_r, dst_r = slot_refs(si)
                    pltpu.make_async_copy(src_r, dst_r, sem_list[si]).wait()

            @pl.when(live)
            def _():
                reduce_store(s3, t - (NSLOT - 1))

            pw1 = pw + 1
            wrap = pw1 == 7
            return (jnp.where(wrap, ph + 1, ph), jnp.where(wrap, 0, pw1))

        lax.fori_loop(0, NBINS, bin_step, (i32(0), i32(0)))
        for j in (NBINS - 3, NBINS - 2, NBINS - 1):
            si = j & (NSLOT - 1)
            src_r, dst_r = slot_refs(si)
            pltpu.make_async_copy(src_r, dst_r, sem_list[si]).wait()
            reduce_store(si, j)

        pltpu.sync_copy(out_roi_v, out_hbm.at[pl.ds((roi0 + r) * ROI_STRIDE, ROI_STRIDE)])
        return carry0

    lax.fori_loop(0, ROIS_PER_TILE, roi_body, 0)


def _make_sc_pool():
    mesh = plsc.VectorSubcoreMesh(core_axis_name="c", subcore_axis_name="s")
    return pl.kernel(
        _sc_pool_body,
        mesh=mesh,
        out_type=jax.ShapeDtypeStruct((N_ROIS * ROI_STRIDE, C), f32),
        scratch_types=[
            pltpu.VMEM((ROIS_PER_TILE, 16), f32),
            pltpu.VMEM((ROIS_PER_TILE, 128), f32),
            pltpu.VMEM((NSLOT * 16, C), f32),
            pltpu.VMEM((16,), i32),
            pltpu.VMEM((16,), i32),
            pltpu.VMEM((16,), i32),
            pltpu.VMEM((16,), i32),
            pltpu.VMEM((NSLOT, 16), f32),
            pltpu.VMEM((ROI_STRIDE, C), f32),
            pltpu.SemaphoreType.DMA,
            pltpu.SemaphoreType.DMA,
            pltpu.SemaphoreType.DMA,
            pltpu.SemaphoreType.DMA,
        ],
    )


MM_K = ROI_STRIDE * C


def _mm_body(a_ref, w_ref, b_ref, o_ref):
    o_ref[...] = jnp.dot(a_ref[...], w_ref[...],
                         preferred_element_type=f32) + b_ref[...]


def _offset_matmul(a, w_pad, b_pad):
    return pl.pallas_call(
        _mm_body,
        out_shape=jax.ShapeDtypeStruct((N_ROIS, 128), f32),
    )(a, w_pad, b_pad)


def kernel(featuremap, rois, W_fc, b_fc):
    fm_rows = featuremap.reshape(B * HF * WF, C)
    rois_pad = jnp.pad(rois, ((0, 0), (0, 11)))
    w_big = jnp.zeros((ROI_STRIDE, C, 128), f32).at[:NBINS, :, :98].set(
        W_fc.reshape(NBINS, C, 98)).reshape(MM_K, 128)
    b_pad = jnp.pad(b_fc, ((0, 30),)).reshape(1, 128)

    sc_pool = _make_sc_pool()

    off0 = jnp.zeros((N_ROIS, 128), f32)
    out1 = sc_pool(fm_rows, rois_pad, off0)
    off = _offset_matmul(out1.reshape(N_ROIS, MM_K), w_big, b_pad)
    out2 = sc_pool(fm_rows, rois_pad, off)
    return out2.reshape(N_ROIS, ROI_STRIDE, C)[:, :NBINS].reshape(
        N_ROIS, POOLED, POOLED, C)


# speedup vs baseline: 1.0497x; 1.0008x over previous
import functools

import jax
import jax.numpy as jnp
from jax import lax
from jax.experimental import pallas as pl
from jax.experimental.pallas import tpu as pltpu
from jax.experimental.pallas import tpu_sc as plsc

B = 2
HF = 64
WF = 64
C = 256
N_ROIS = 512
POOLED = 7
SPP = 4
SCALE = 0.0625
NBINS = POOLED * POOLED
ROI_STRIDE = 56

NUM_TILES = 32
ROIS_PER_TILE = N_ROIS // NUM_TILES

f32 = jnp.float32
i32 = jnp.int32


def _recip04(c):
    return jnp.where(c == 1.0, f32(1.0),
           jnp.where(c == 2.0, f32(0.5),
           jnp.where(c == 3.0, f32(1.0 / 3.0),
           jnp.where(c == 4.0, f32(0.25), f32(0.0)))))


def _floor_i32(x):
    ti = x.astype(i32)
    return ti - (ti.astype(f32) > x).astype(i32)


def _round_half_even(x):
    ti = _floor_i32(x)
    t = ti.astype(f32)
    f = x - t
    odd = (ti % 2) == 1
    return jnp.where(f > 0.5, t + 1.0, jnp.where(f < 0.5, t, jnp.where(odd, t + 1.0, t)))


NSLOT = 4


def _sc_pool_body(fm_hbm, rois_hbm, off_hbm, out_hbm,
                  rois_v, off_v, patch_v, idx0_v, idx1_v, idx2_v, idx3_v,
                  a_v, out_roi_v, sem0, sem1, sem2, sem3):
    sem_list = (sem0, sem1, sem2, sem3)
    idx_refs = (idx0_v, idx1_v, idx2_v, idx3_v)
    wid = lax.axis_index("s") * 2 + lax.axis_index("c")
    roi0 = wid * ROIS_PER_TILE

    pltpu.sync_copy(rois_hbm.at[pl.ds(roi0, ROIS_PER_TILE)], rois_v)
    pltpu.sync_copy(off_hbm.at[pl.ds(roi0, ROIS_PER_TILE)], off_v)

    io = lax.iota(i32, 16)
    khi = lax.shift_right_logical(io, 2)
    kwi = lax.bitwise_and(io, 3)
    khf = khi.astype(f32)
    kwf = kwi.astype(f32)
    lane_off = khi * WF + kwi

    def slot_refs(si):
        return (fm_hbm.at[idx_refs[si]],
                patch_v.at[pl.ds(si * 16, 16)])

    def reduce_store(s, binidx):
        av = a_v[s, pl.ds(0, 16)]
        a_s = [av[p] for p in range(16)]
        base = s * 16
        for cvb in range(16):
            col = pl.ds(cvb * 16, 16)
            s0 = a_s[0] * patch_v[base, col]
            s1 = a_s[1] * patch_v[base + 1, col]
            s2 = a_s[2] * patch_v[base + 2, col]
            s3 = a_s[3] * patch_v[base + 3, col]
            for p in range(4, 16, 4):
                s0 = s0 + a_s[p] * patch_v[base + p, col]
                s1 = s1 + a_s[p + 1] * patch_v[base + p + 1, col]
                s2 = s2 + a_s[p + 2] * patch_v[base + p + 2, col]
                s3 = s3 + a_s[p + 3] * patch_v[base + p + 3, col]
            out_roi_v[binidx, col] = (s0 + s1) + (s2 + s3)

    z16v = jnp.zeros((16,), f32)
    for pr in range(NBINS, ROI_STRIDE):
        for cvb in range(16):
            out_roi_v[pr, pl.ds(cvb * 16, 16)] = z16v

    def roi_body(r, carry0):
        rv = rois_v[r, pl.ds(0, 16)]
        b_i = rv[0].astype(i32)
        x1 = _round_half_even(rv[1])
        y1 = _round_half_even(rv[2])
        x2 = _round_half_even(rv[3])
        y2 = _round_half_even(rv[4])
        rsw = x1 * SCALE - 0.5
        rsh = y1 * SCALE - 0.5
        rew = (x2 + 1.0) * SCALE - 0.5
        reh = (y2 + 1.0) * SCALE - 0.5
        roi_w = jnp.maximum(rew - rsw, 0.1)
        roi_h = jnp.maximum(reh - rsh, 0.1)
        bin_w = roi_w * f32(1.0 / 7.0)
        bin_h = roi_h * f32(1.0 / 7.0)
        sub_w = bin_w * 0.25
        sub_h = bin_h * 0.25
        base_b = b_i * (HF * WF)

        def bin_step(t, carry):
            ph, pw = carry
            binidx = ph * 7 + pw
            tx = off_v[r, pl.ds(binidx, 16)][0]
            ty = off_v[r, pl.ds(49 + binidx, 16)][0]
            pwf = pw.astype(f32)
            phf = ph.astype(f32)
            wstart = pwf * bin_w + rsw + tx * roi_w
            hstart = phf * bin_h + rsh + ty * roi_h

            w0i = jnp.minimum(_floor_i32(jnp.minimum(jnp.maximum(wstart, 0.0), 63.0)), 60)
            h0i = jnp.minimum(_floor_i32(jnp.minimum(jnp.maximum(hstart, 0.0), 63.0)), 60)
            w0f = w0i.astype(f32)
            h0f = h0i.astype(f32)

            idxvec = (base_b + h0i * WF + w0i) + lane_off
            slot = lax.bitwise_and(t, NSLOT - 1)
            for si in range(NSLOT):
                @pl.when(slot == si)
                def _(si=si):
                    idx_refs[si][...] = idxvec
                    src_r, dst_r = slot_refs(si)
                    pltpu.async_copy(src_r, dst_r, sem_list[si])

            Ww = jnp.zeros((16,), f32)
            Wh = jnp.zeros((16,), f32)
            cw = f32(0.0)
            ch = f32(0.0)
            for i in range(SPP):
                w = wstart + i * sub_w
                mw = jnp.where((w >= -0.5) & (w <= 63.5), f32(1.0), f32(0.0))
                wc = jnp.minimum(jnp.maximum(w, 0.0), 63.0)
                Ww = Ww + mw * jnp.maximum(1.0 - jnp.abs(wc - (w0f + kwf)), 0.0)
                cw = cw + mw
                h = hstart + i * sub_h
                mh = jnp.where((h >= -0.5) & (h <= 63.5), f32(1.0), f32(0.0))
                hc = jnp.minimum(jnp.maximum(h, 0.0), 63.0)
                Wh = Wh + mh * jnp.maximum(1.0 - jnp.abs(hc - (h0f + khf)), 0.0)
                ch = ch + mh

            scale = _recip04(cw) * _recip04(ch)
            a_v[slot, pl.ds(0, 16)] = Wh * Ww * scale

            s3 = lax.bitwise_and(t + 1, NSLOT - 1)
            live = t >= NSLOT - 1
            for si in range(NSLOT):
                @pl.when(live & (s3 == si))
                def _(si=si):
                    src_r, dst_r = slot_refs(si)
                    pltpu.make_async_copy(src_r, dst_r, sem_list[si]).wait()

            @pl.when(live)
            def _():
                reduce_store(s3, t - (NSLOT - 1))

            pw1 = pw + 1
            wrap = pw1 == 7
            return (jnp.where(wrap, ph + 1, ph), jnp.where(wrap, 0, pw1))

        lax.fori_loop(0, NBINS, bin_step, (i32(0), i32(0)))
        for j in (NBINS - 3, NBINS - 2, NBINS - 1):
            si = j & (NSLOT - 1)
            src_r, dst_r = slot_refs(si)
            pltpu.make_async_copy(src_r, dst_r, sem_list[si]).wait()
            reduce_store(si, j)

        pltpu.sync_copy(out_roi_v, out_hbm.at[pl.ds((roi0 + r) * ROI_STRIDE, ROI_STRIDE)])
        return carry0

    lax.fori_loop(0, ROIS_PER_TILE, roi_body, 0)


def _make_sc_pool():
    mesh = plsc.VectorSubcoreMesh(core_axis_name="c", subcore_axis_name="s")
    return pl.kernel(
        _sc_pool_body,
        mesh=mesh,
        out_type=jax.ShapeDtypeStruct((N_ROIS * ROI_STRIDE, C), f32),
        scratch_types=[
            pltpu.VMEM((ROIS_PER_TILE, 16), f32),
            pltpu.VMEM((ROIS_PER_TILE, 128), f32),
            pltpu.VMEM((NSLOT * 16, C), f32),
            pltpu.VMEM((16,), i32),
            pltpu.VMEM((16,), i32),
            pltpu.VMEM((16,), i32),
            pltpu.VMEM((16,), i32),
            pltpu.VMEM((NSLOT, 16), f32),
            pltpu.VMEM((ROI_STRIDE, C), f32),
            pltpu.SemaphoreType.DMA,
            pltpu.SemaphoreType.DMA,
            pltpu.SemaphoreType.DMA,
            pltpu.SemaphoreType.DMA,
        ],
    )


MM_K = ROI_STRIDE * C


def _mm_body(a_ref, w_ref, b_ref, o_ref):
    o_ref[...] = jnp.dot(a_ref[...], w_ref[...],
                         preferred_element_type=f32) + b_ref[...]


def _offset_matmul(a, w_pad, b_pad):
    return pl.pallas_call(
        _mm_body,
        out_shape=jax.ShapeDtypeStruct((N_ROIS, 128), f32),
    )(a, w_pad, b_pad)


def kernel(featuremap, rois, W_fc, b_fc):
    fm_rows = featuremap.reshape(B * HF * WF, C)
    rois_pad = jnp.pad(rois, ((0, 0), (0, 11)))
    w_big = jnp.zeros((ROI_STRIDE, C, 128), f32).at[:NBINS, :, :98].set(
        W_fc.reshape(NBINS, C, 98)).reshape(MM_K, 128)
    b_pad = jnp.pad(b_fc, ((0, 30),)).reshape(1, 128)

    sc_pool = _make_sc_pool()

    off0 = jnp.zeros((N_ROIS, 128), f32)
    out1 = sc_pool(fm_rows, rois_pad, off0)
    off = _offset_matmul(out1.reshape(N_ROIS, MM_K), w_big, b_pad)
    out2 = sc_pool(fm_rows, rois_pad, off)
    return out2.reshape(N_ROIS, ROI_STRIDE, C)[:, :NBINS].reshape(
        N_ROIS, POOLED, POOLED, C)


# speedup vs baseline: 1.0836x; 1.0323x over previous
import functools

import jax
import jax.numpy as jnp
from jax import lax
from jax.experimental import pallas as pl
from jax.experimental.pallas import tpu as pltpu
from jax.experimental.pallas import tpu_sc as plsc

B = 2
HF = 64
WF = 64
C = 256
N_ROIS = 512
POOLED = 7
SPP = 4
SCALE = 0.0625
NBINS = POOLED * POOLED
ROI_STRIDE = 56

NUM_TILES = 32
ROIS_PER_TILE = N_ROIS // NUM_TILES

f32 = jnp.float32
i32 = jnp.int32


def _recip04(c):
    return jnp.where(c == 1.0, f32(1.0),
           jnp.where(c == 2.0, f32(0.5),
           jnp.where(c == 3.0, f32(1.0 / 3.0),
           jnp.where(c == 4.0, f32(0.25), f32(0.0)))))


def _floor_i32(x):
    ti = x.astype(i32)
    return ti - (ti.astype(f32) > x).astype(i32)


def _round_half_even(x):
    ti = _floor_i32(x)
    t = ti.astype(f32)
    f = x - t
    odd = (ti % 2) == 1
    return jnp.where(f > 0.5, t + 1.0, jnp.where(f < 0.5, t, jnp.where(odd, t + 1.0, t)))


NSLOT = 4


def _sc_pool_body(fm_hbm, rois_hbm, off_hbm, out_hbm,
                  rois_v, off_v, patch_v, idx0_v, idx1_v, idx2_v, idx3_v,
                  a_v, out_roi_v, sem0, sem1, sem2, sem3):
    sem_list = (sem0, sem1, sem2, sem3)
    idx_refs = (idx0_v, idx1_v, idx2_v, idx3_v)
    wid = lax.axis_index("s") * 2 + lax.axis_index("c")
    roi0 = wid * ROIS_PER_TILE

    pltpu.sync_copy(rois_hbm.at[pl.ds(roi0, ROIS_PER_TILE)], rois_v)
    pltpu.sync_copy(off_hbm.at[pl.ds(roi0, ROIS_PER_TILE)], off_v)

    io = lax.iota(i32, 16)
    khi = lax.shift_right_logical(io, 2)
    kwi = lax.bitwise_and(io, 3)
    khf = khi.astype(f32)
    kwf = kwi.astype(f32)
    lane_off = khi * WF + kwi


    def slot_refs(si):
        return (fm_hbm.at[idx_refs[si]],
                patch_v.at[pl.ds(si * 16, 16)])

    def reduce_px(s, binidx, av, pixels):
        a_s = [av[p] for p in pixels]
        base = s * 16
        for cvb in range(16):
            col = pl.ds(cvb * 16, 16)
            acc = [a_s[k] * patch_v[base + p, col]
                   for k, p in enumerate(pixels[:4])]
            for k, p in enumerate(pixels[4:]):
                acc[k % 4] = acc[k % 4] + a_s[4 + k] * patch_v[base + p, col]
            tot = (acc[0] + acc[1]) + (acc[2] + acc[3])
            out_roi_v[binidx, col] = tot

    def reduce_store(s, binidx, variants=False):
        av = a_v[s, pl.ds(0, 16)]
        if not variants:
            reduce_px(s, binidx, av, list(range(16)))
            return
        edge_sum = ((av[3] + av[7]) + (av[11] + av[12])) + ((av[13] + av[14]) + av[15])
        small = edge_sum == 0.0

        @pl.when(small)
        def _():
            reduce_px(s, binidx, av, [0, 1, 2, 4, 5, 6, 8, 9, 10])

        @pl.when(jnp.logical_not(small))
        def _():
            reduce_px(s, binidx, av, list(range(16)))

    z16v = jnp.zeros((16,), f32)
    for pr in range(NBINS, ROI_STRIDE):
        for cvb in range(16):
            out_roi_v[pr, pl.ds(cvb * 16, 16)] = z16v

    def roi_body(r, carry0):
        rv = rois_v[r, pl.ds(0, 16)]
        b_i = rv[0].astype(i32)
        x1 = _round_half_even(rv[1])
        y1 = _round_half_even(rv[2])
        x2 = _round_half_even(rv[3])
        y2 = _round_half_even(rv[4])
        rsw = x1 * SCALE - 0.5
        rsh = y1 * SCALE - 0.5
        rew = (x2 + 1.0) * SCALE - 0.5
        reh = (y2 + 1.0) * SCALE - 0.5
        roi_w = jnp.maximum(rew - rsw, 0.1)
        roi_h = jnp.maximum(reh - rsh, 0.1)
        bin_w = roi_w * f32(1.0 / 7.0)
        bin_h = roi_h * f32(1.0 / 7.0)
        sub_w = bin_w * 0.25
        sub_h = bin_h * 0.25
        base_b = b_i * (HF * WF)

        def bin_step(t, carry):
            ph, pw = carry
            binidx = ph * 7 + pw
            tx = off_v[r, pl.ds(binidx, 16)][0]
            ty = off_v[r, pl.ds(49 + binidx, 16)][0]
            pwf = pw.astype(f32)
            phf = ph.astype(f32)
            wstart = pwf * bin_w + rsw + tx * roi_w
            hstart = phf * bin_h + rsh + ty * roi_h

            w0i = jnp.minimum(_floor_i32(jnp.minimum(jnp.maximum(wstart, 0.0), 63.0)), 60)
            h0i = jnp.minimum(_floor_i32(jnp.minimum(jnp.maximum(hstart, 0.0), 63.0)), 60)
            w0f = w0i.astype(f32)
            h0f = h0i.astype(f32)

            idxvec = (base_b + h0i * WF + w0i) + lane_off
            slot = lax.bitwise_and(t, NSLOT - 1)
            for si in range(NSLOT):
                @pl.when(slot == si)
                def _(si=si):
                    idx_refs[si][...] = idxvec
                    src_r, dst_r = slot_refs(si)
                    pltpu.async_copy(src_r, dst_r, sem_list[si])

            Ww = jnp.zeros((16,), f32)
            Wh = jnp.zeros((16,), f32)
            cw = f32(0.0)
            ch = f32(0.0)
            for i in range(SPP):
                w = wstart + i * sub_w
                mw = jnp.where((w >= -0.5) & (w <= 63.5), f32(1.0), f32(0.0))
                wc = jnp.minimum(jnp.maximum(w, 0.0), 63.0)
                Ww = Ww + mw * jnp.maximum(1.0 - jnp.abs(wc - (w0f + kwf)), 0.0)
                cw = cw + mw
                h = hstart + i * sub_h
                mh = jnp.where((h >= -0.5) & (h <= 63.5), f32(1.0), f32(0.0))
                hc = jnp.minimum(jnp.maximum(h, 0.0), 63.0)
                Wh = Wh + mh * jnp.maximum(1.0 - jnp.abs(hc - (h0f + khf)), 0.0)
                ch = ch + mh

            scale = _recip04(cw) * _recip04(ch)
            a_v[slot, pl.ds(0, 16)] = Wh * Ww * scale

            s3 = lax.bitwise_and(t + 1, NSLOT - 1)
            live = t >= NSLOT - 1
            for si in range(NSLOT):
                @pl.when(live & (s3 == si))
                def _(si=si):
                    src_r, dst_r = slot_refs(si)
                    pltpu.make_async_copy(src_r, dst_r, sem_list[si]).wait()

            @pl.when(live)
            def _():
                reduce_store(s3, t - (NSLOT - 1), variants=True)

            pw1 = pw + 1
            wrap = pw1 == 7
            return (jnp.where(wrap, ph + 1, ph), jnp.where(wrap, 0, pw1))

        lax.fori_loop(0, NBINS, bin_step, (i32(0), i32(0)))
        for j in (NBINS - 3, NBINS - 2, NBINS - 1):
            si = j & (NSLOT - 1)
            src_r, dst_r = slot_refs(si)
            pltpu.make_async_copy(src_r, dst_r, sem_list[si]).wait()
            reduce_store(si, j)

        pltpu.sync_copy(out_roi_v, out_hbm.at[pl.ds((roi0 + r) * ROI_STRIDE, ROI_STRIDE)])
        return carry0

    lax.fori_loop(0, ROIS_PER_TILE, roi_body, 0)


def _make_sc_pool():
    mesh = plsc.VectorSubcoreMesh(core_axis_name="c", subcore_axis_name="s")
    return pl.kernel(
        _sc_pool_body,
        mesh=mesh,
        out_type=jax.ShapeDtypeStruct((N_ROIS * ROI_STRIDE, C), f32),
        scratch_types=[
            pltpu.VMEM((ROIS_PER_TILE, 16), f32),
            pltpu.VMEM((ROIS_PER_TILE, 128), f32),
            pltpu.VMEM((NSLOT * 16, C), f32),
            pltpu.VMEM((16,), i32),
            pltpu.VMEM((16,), i32),
            pltpu.VMEM((16,), i32),
            pltpu.VMEM((16,), i32),
            pltpu.VMEM((NSLOT, 16), f32),
            pltpu.VMEM((ROI_STRIDE, C), f32),
            pltpu.SemaphoreType.DMA,
            pltpu.SemaphoreType.DMA,
            pltpu.SemaphoreType.DMA,
            pltpu.SemaphoreType.DMA,
        ],
    )


MM_K = ROI_STRIDE * C


def _mm_body(a_ref, w_ref, b_ref, o_ref):
    o_ref[...] = jnp.dot(a_ref[...], w_ref[...],
                         preferred_element_type=f32) + b_ref[...]


def _offset_matmul(a, w_pad, b_pad):
    return pl.pallas_call(
        _mm_body,
        out_shape=jax.ShapeDtypeStruct((N_ROIS, 128), f32),
    )(a, w_pad, b_pad)


def kernel(featuremap, rois, W_fc, b_fc):
    fm_rows = featuremap.reshape(B * HF * WF, C)
    rois_pad = jnp.pad(rois, ((0, 0), (0, 11)))
    w_big = jnp.zeros((ROI_STRIDE, C, 128), f32).at[:NBINS, :, :98].set(
        W_fc.reshape(NBINS, C, 98)).reshape(MM_K, 128)
    b_pad = jnp.pad(b_fc, ((0, 30),)).reshape(1, 128)

    sc_pool = _make_sc_pool()

    off0 = jnp.zeros((N_ROIS, 128), f32)
    out1 = sc_pool(fm_rows, rois_pad, off0)
    off = _offset_matmul(out1.reshape(N_ROIS, MM_K), w_big, b_pad)
    out2 = sc_pool(fm_rows, rois_pad, off)
    return out2.reshape(N_ROIS, ROI_STRIDE, C)[:, :NBINS].reshape(
        N_ROIS, POOLED, POOLED, C)


# speedup vs baseline: 1.0903x; 1.0062x over previous
import functools

import jax
import jax.numpy as jnp
from jax import lax
from jax.experimental import pallas as pl
from jax.experimental.pallas import tpu as pltpu
from jax.experimental.pallas import tpu_sc as plsc

B = 2
HF = 64
WF = 64
C = 256
N_ROIS = 512
POOLED = 7
SPP = 4
SCALE = 0.0625
NBINS = POOLED * POOLED
ROI_STRIDE = 56

NUM_TILES = 32
ROIS_PER_TILE = N_ROIS // NUM_TILES

f32 = jnp.float32
i32 = jnp.int32


def _recip04(c):
    return jnp.where(c == 1.0, f32(1.0),
           jnp.where(c == 2.0, f32(0.5),
           jnp.where(c == 3.0, f32(1.0 / 3.0),
           jnp.where(c == 4.0, f32(0.25), f32(0.0)))))


def _floor_i32(x):
    ti = x.astype(i32)
    return ti - (ti.astype(f32) > x).astype(i32)


def _round_half_even(x):
    ti = _floor_i32(x)
    t = ti.astype(f32)
    f = x - t
    odd = (ti % 2) == 1
    return jnp.where(f > 0.5, t + 1.0, jnp.where(f < 0.5, t, jnp.where(odd, t + 1.0, t)))


NSLOT = 4


def _sc_pool_body(fm_hbm, rois_hbm, off_hbm, out_hbm,
                  rois_v, off_v, patch_v, idx0_v, idx1_v, idx2_v, idx3_v,
                  a_v, out_roi_v, sem0, sem1, sem2, sem3, osem0, osem1):
    osem_list = (osem0, osem1)
    sem_list = (sem0, sem1, sem2, sem3)
    idx_refs = (idx0_v, idx1_v, idx2_v, idx3_v)
    wid = lax.axis_index("s") * 2 + lax.axis_index("c")
    roi0 = wid * ROIS_PER_TILE

    pltpu.sync_copy(rois_hbm.at[pl.ds(roi0, ROIS_PER_TILE)], rois_v)
    pltpu.sync_copy(off_hbm.at[pl.ds(roi0, ROIS_PER_TILE)], off_v)

    io = lax.iota(i32, 16)
    khi = lax.shift_right_logical(io, 2)
    kwi = lax.bitwise_and(io, 3)
    khf = khi.astype(f32)
    kwf = kwi.astype(f32)
    lane_off = khi * WF + kwi


    def slot_refs(si):
        return (fm_hbm.at[idx_refs[si]],
                patch_v.at[pl.ds(si * 16, 16)])

    def reduce_px(s, binidx, av, pixels):
        a_s = [av[p] for p in pixels]
        base = s * 16
        for cvb in range(16):
            col = pl.ds(cvb * 16, 16)
            acc = [a_s[k] * patch_v[base + p, col]
                   for k, p in enumerate(pixels[:4])]
            for k, p in enumerate(pixels[4:]):
                acc[k % 4] = acc[k % 4] + a_s[4 + k] * patch_v[base + p, col]
            tot = (acc[0] + acc[1]) + (acc[2] + acc[3])
            out_roi_v[binidx, col] = tot

    def reduce_store(s, binidx, variants=False):
        av = a_v[s, pl.ds(0, 16)]
        if not variants:
            reduce_px(s, binidx, av, list(range(16)))
            return
        edge_sum = ((av[3] + av[7]) + (av[11] + av[12])) + ((av[13] + av[14]) + av[15])
        small = edge_sum == 0.0

        @pl.when(small)
        def _():
            reduce_px(s, binidx, av, [0, 1, 2, 4, 5, 6, 8, 9, 10])

        @pl.when(jnp.logical_not(small))
        def _():
            reduce_px(s, binidx, av, list(range(16)))

    z16v = jnp.zeros((16,), f32)
    for half in range(2):
        for pr in range(NBINS, ROI_STRIDE):
            for cvb in range(16):
                out_roi_v[half * ROI_STRIDE + pr, pl.ds(cvb * 16, 16)] = z16v

    def out_half_refs(hf, r_static_expr):
        return (out_roi_v.at[pl.ds(hf * ROI_STRIDE, ROI_STRIDE)],)

    def roi_body(r, carry0):
        half = lax.bitwise_and(r, 1)
        hbase = half * ROI_STRIDE
        @pl.when(r >= 2)
        def _():
            for hf in range(2):
                @pl.when(half == hf)
                def _(hf=hf):
                    src_r = out_roi_v.at[pl.ds(hf * ROI_STRIDE, ROI_STRIDE)]
                    dst_r = out_hbm.at[pl.ds((roi0 + r - 2) * ROI_STRIDE, ROI_STRIDE)]
                    pltpu.make_async_copy(src_r, dst_r, osem_list[hf]).wait()
        rv = rois_v[r, pl.ds(0, 16)]
        b_i = rv[0].astype(i32)
        x1 = _round_half_even(rv[1])
        y1 = _round_half_even(rv[2])
        x2 = _round_half_even(rv[3])
        y2 = _round_half_even(rv[4])
        rsw = x1 * SCALE - 0.5
        rsh = y1 * SCALE - 0.5
        rew = (x2 + 1.0) * SCALE - 0.5
        reh = (y2 + 1.0) * SCALE - 0.5
        roi_w = jnp.maximum(rew - rsw, 0.1)
        roi_h = jnp.maximum(reh - rsh, 0.1)
        bin_w = roi_w * f32(1.0 / 7.0)
        bin_h = roi_h * f32(1.0 / 7.0)
        sub_w = bin_w * 0.25
        sub_h = bin_h * 0.25
        base_b = b_i * (HF * WF)

        def bin_step(t, carry):
            ph, pw = carry
            binidx = ph * 7 + pw
            tx = off_v[r, pl.ds(binidx, 16)][0]
            ty = off_v[r, pl.ds(49 + binidx, 16)][0]
            pwf = pw.astype(f32)
            phf = ph.astype(f32)
            wstart = pwf * bin_w + rsw + tx * roi_w
            hstart = phf * bin_h + rsh + ty * roi_h

            w0i = jnp.minimum(_floor_i32(jnp.minimum(jnp.maximum(wstart, 0.0), 63.0)), 60)
            h0i = jnp.minimum(_floor_i32(jnp.minimum(jnp.maximum(hstart, 0.0), 63.0)), 60)
            w0f = w0i.astype(f32)
            h0f = h0i.astype(f32)

            idxvec = (base_b + h0i * WF + w0i) + lane_off
            slot = lax.bitwise_and(t, NSLOT - 1)
            for si in range(NSLOT):
                @pl.when(slot == si)
                def _(si=si):
                    idx_refs[si][...] = idxvec
                    src_r, dst_r = slot_refs(si)
                    pltpu.async_copy(src_r, dst_r, sem_list[si])

            Ww = jnp.zeros((16,), f32)
            Wh = jnp.zeros((16,), f32)
            cw = f32(0.0)
            ch = f32(0.0)
            for i in range(SPP):
                w = wstart + i * sub_w
                mw = jnp.where((w >= -0.5) & (w <= 63.5), f32(1.0), f32(0.0))
                wc = jnp.minimum(jnp.maximum(w, 0.0), 63.0)
                Ww = Ww + mw * jnp.maximum(1.0 - jnp.abs(wc - (w0f + kwf)), 0.0)
                cw = cw + mw
                h = hstart + i * sub_h
                mh = jnp.where((h >= -0.5) & (h <= 63.5), f32(1.0), f32(0.0))
                hc = jnp.minimum(jnp.maximum(h, 0.0), 63.0)
                Wh = Wh + mh * jnp.maximum(1.0 - jnp.abs(hc - (h0f + khf)), 0.0)
                ch = ch + mh

            scale = _recip04(cw) * _recip04(ch)
            a_v[slot, pl.ds(0, 16)] = Wh * Ww * scale

            s3 = lax.bitwise_and(t + 1, NSLOT - 1)
            live = t >= NSLOT - 1
            for si in range(NSLOT):
                @pl.when(live & (s3 == si))
                def _(si=si):
                    src_r, dst_r = slot_refs(si)
                    pltpu.make_async_copy(src_r, dst_r, sem_list[si]).wait()

            @pl.when(live)
            def _():
                reduce_store(s3, hbase + (t - (NSLOT - 1)), variants=True)

            pw1 = pw + 1
            wrap = pw1 == 7
            return (jnp.where(wrap, ph + 1, ph), jnp.where(wrap, 0, pw1))

        lax.fori_loop(0, NBINS, bin_step, (i32(0), i32(0)))
        for j in (NBINS - 3, NBINS - 2, NBINS - 1):
            si = j & (NSLOT - 1)
            src_r, dst_r = slot_refs(si)
            pltpu.make_async_copy(src_r, dst_r, sem_list[si]).wait()
            reduce_store(si, hbase + j)

        for hf in range(2):
            @pl.when(half == hf)
            def _(hf=hf):
                src_r = out_roi_v.at[pl.ds(hf * ROI_STRIDE, ROI_STRIDE)]
                dst_r = out_hbm.at[pl.ds((roi0 + r) * ROI_STRIDE, ROI_STRIDE)]
                pltpu.async_copy(src_r, dst_r, osem_list[hf])
        return carry0

    lax.fori_loop(0, ROIS_PER_TILE, roi_body, 0)
    for rr in (ROIS_PER_TILE - 2, ROIS_PER_TILE - 1):
        hf = rr & 1
        src_r = out_roi_v.at[pl.ds(hf * ROI_STRIDE, ROI_STRIDE)]
        dst_r = out_hbm.at[pl.ds((roi0 + rr) * ROI_STRIDE, ROI_STRIDE)]
        pltpu.make_async_copy(src_r, dst_r, osem_list[hf]).wait()


def _make_sc_pool():
    mesh = plsc.VectorSubcoreMesh(core_axis_name="c", subcore_axis_name="s")
    return pl.kernel(
        _sc_pool_body,
        mesh=mesh,
        out_type=jax.ShapeDtypeStruct((N_ROIS * ROI_STRIDE, C), f32),
        scratch_types=[
            pltpu.VMEM((ROIS_PER_TILE, 16), f32),
            pltpu.VMEM((ROIS_PER_TILE, 128), f32),
            pltpu.VMEM((NSLOT * 16, C), f32),
            pltpu.VMEM((16,), i32),
            pltpu.VMEM((16,), i32),
            pltpu.VMEM((16,), i32),
            pltpu.VMEM((16,), i32),
            pltpu.VMEM((NSLOT, 16), f32),
            pltpu.VMEM((2 * ROI_STRIDE, C), f32),
            pltpu.SemaphoreType.DMA,
            pltpu.SemaphoreType.DMA,
            pltpu.SemaphoreType.DMA,
            pltpu.SemaphoreType.DMA,
            pltpu.SemaphoreType.DMA,
            pltpu.SemaphoreType.DMA,
        ],
    )


MM_K = ROI_STRIDE * C


def _mm_body(a_ref, w_ref, b_ref, o_ref):
    o_ref[...] = jnp.dot(a_ref[...], w_ref[...],
                         preferred_element_type=f32) + b_ref[...]


def _offset_matmul(a, w_pad, b_pad):
    return pl.pallas_call(
        _mm_body,
        out_shape=jax.ShapeDtypeStruct((N_ROIS, 128), f32),
    )(a, w_pad, b_pad)


def kernel(featuremap, rois, W_fc, b_fc):
    fm_rows = featuremap.reshape(B * HF * WF, C)
    rois_pad = jnp.pad(rois, ((0, 0), (0, 11)))
    w_big = jnp.zeros((ROI_STRIDE, C, 128), f32).at[:NBINS, :, :98].set(
        W_fc.reshape(NBINS, C, 98)).reshape(MM_K, 128)
    b_pad = jnp.pad(b_fc, ((0, 30),)).reshape(1, 128)

    sc_pool = _make_sc_pool()

    off0 = jnp.zeros((N_ROIS, 128), f32)
    out1 = sc_pool(fm_rows, rois_pad, off0)
    off = _offset_matmul(out1.reshape(N_ROIS, MM_K), w_big, b_pad)
    out2 = sc_pool(fm_rows, rois_pad, off)
    return out2.reshape(N_ROIS, ROI_STRIDE, C)[:, :NBINS].reshape(
        N_ROIS, POOLED, POOLED, C)
